# baseline (speedup 1.0000x reference)
import functools

import jax
import jax.numpy as jnp
from jax import lax
from jax.experimental import pallas as pl
from jax.experimental.pallas import tpu as pltpu
from jax.experimental.pallas import tpu_sc as plsc

B = 1024
D = 32


def _make_sc_gather():
    info = plsc.get_sparse_core_info()
    nc, ns = info.num_cores, info.num_subcores
    nw = nc * ns
    bpw = B // nw
    mesh = plsc.VectorSubcoreMesh(core_axis_name="c", subcore_axis_name="s")

    @functools.partial(
        pl.kernel,
        out_type=(
            jax.ShapeDtypeStruct((B, D), jnp.float32),
            jax.ShapeDtypeStruct((B, D), jnp.float32),
            jax.ShapeDtypeStruct((B,), jnp.float32),
            jax.ShapeDtypeStruct((B,), jnp.float32),
        ),
        mesh=mesh,
        compiler_params=pltpu.CompilerParams(use_tc_tiling_on_sc=False),
        scratch_types=[
            pltpu.VMEM((bpw,), jnp.int32),
            pltpu.VMEM((bpw,), jnp.int32),
            pltpu.VMEM((bpw, D), jnp.float32),
            pltpu.VMEM((bpw, D), jnp.float32),
            pltpu.VMEM((bpw,), jnp.float32),
            pltpu.VMEM((bpw,), jnp.float32),
            pltpu.SemaphoreType.DMA,
            pltpu.SemaphoreType.DMA,
            pltpu.SemaphoreType.DMA,
            pltpu.SemaphoreType.DMA,
        ],
    )
    def gather_kernel(
        user_emb_hbm,
        item_emb_hbm,
        user_bias_hbm,
        item_bias_hbm,
        uids_hbm,
        iids_hbm,
        u_out,
        q_out,
        ub_out,
        ib_out,
        uidx_v,
        iidx_v,
        urows_v,
        qrows_v,
        ubr_v,
        ibr_v,
        sem_u,
        sem_q,
        sem_ub,
        sem_ib,
    ):
        wid = lax.axis_index("s") * nc + lax.axis_index("c")
        base = wid * bpw
        pltpu.sync_copy(uids_hbm.at[pl.ds(base, bpw)], uidx_v)
        pltpu.sync_copy(iids_hbm.at[pl.ds(base, bpw)], iidx_v)
        cu = pltpu.async_copy(user_emb_hbm.at[uidx_v], urows_v, sem_u)
        cq = pltpu.async_copy(item_emb_hbm.at[iidx_v], qrows_v, sem_q)
        cub = pltpu.async_copy(user_bias_hbm.at[uidx_v], ubr_v, sem_ub)
        cib = pltpu.async_copy(item_bias_hbm.at[iidx_v], ibr_v, sem_ib)
        cu.wait()
        cq.wait()
        cub.wait()
        cib.wait()
        pltpu.sync_copy(urows_v, u_out.at[pl.ds(base, bpw)])
        pltpu.sync_copy(qrows_v, q_out.at[pl.ds(base, bpw)])
        pltpu.sync_copy(ubr_v, ub_out.at[pl.ds(base, bpw)])
        pltpu.sync_copy(ibr_v, ib_out.at[pl.ds(base, bpw)])

    return gather_kernel


def _tc_body(u_ref, q_ref, ub_ref, ib_ref, w1t_ref, b1_ref, w2t_ref, b2_ref,
             preds_ref, score_ref):
    u = u_ref[...]
    q = q_ref[...]
    uq = u * q
    bias_col = ub_ref[...] + ib_ref[...]
    ones = jnp.ones((B, D), dtype=jnp.float32)
    preds = lax.dot_general(
        ones, uq, (((1,), (1,)), ((), ())),
        preferred_element_type=jnp.float32,
    )
    preds_ref[...] = preds + bias_col
    cat = jnp.concatenate([u, q, uq], axis=1)
    h = lax.dot_general(
        cat, w1t_ref[...], (((1,), (0,)), ((), ())),
        preferred_element_type=jnp.float32,
    )
    h = jnp.maximum(h + b1_ref[...], 0.0)
    s = lax.dot_general(
        h, w2t_ref[...], (((1,), (0,)), ((), ())),
        preferred_element_type=jnp.float32,
    )
    score_ref[...] = jnp.maximum(s + b2_ref[...], 0.0)


_sc_gather = None


def kernel(user_emb, item_emb, user_bias, item_bias, W1, bias1, W2, bias2,
           user_ids, item_ids):
    global _sc_gather
    if _sc_gather is None:
        _sc_gather = _make_sc_gather()

    u, q, ub, ib = _sc_gather(
        user_emb, item_emb,
        user_bias.reshape(-1), item_bias.reshape(-1),
        user_ids.astype(jnp.int32), item_ids.astype(jnp.int32),
    )
    ub = ub.reshape(B, 1)
    ib = ib.reshape(B, 1)

    w1t = W1.T
    b1 = bias1.reshape(1, 64)
    w2t = W2.T
    b2 = bias2.reshape(1, 1)

    preds, score = pl.pallas_call(
        _tc_body,
        out_shape=(
            jax.ShapeDtypeStruct((B, B), jnp.float32),
            jax.ShapeDtypeStruct((B, 1), jnp.float32),
        ),
    )(u, q, ub, ib, w1t, b1, w2t, b2)
    return (preds, score)


# speedup vs baseline: 1.0023x; 1.0023x over previous
import functools

import jax
import jax.numpy as jnp
from jax import lax
from jax.experimental import pallas as pl
from jax.experimental.pallas import tpu as pltpu
from jax.experimental.pallas import tpu_sc as plsc

B = 1024
D = 32
PK = 4
DW = D * PK


def _make_sc_gather():
    info = plsc.get_sparse_core_info()
    nc, ns = info.num_cores, info.num_subcores
    nw = nc * ns
    bpw = B // nw
    mesh = plsc.VectorSubcoreMesh(core_axis_name="c", subcore_axis_name="s")

    @functools.partial(
        pl.kernel,
        out_type=(
            jax.ShapeDtypeStruct((B, DW), jnp.float32),
            jax.ShapeDtypeStruct((B, DW), jnp.float32),
            jax.ShapeDtypeStruct((B,), jnp.float32),
            jax.ShapeDtypeStruct((B,), jnp.float32),
        ),
        mesh=mesh,
        scratch_types=[
            pltpu.VMEM((bpw,), jnp.int32),
            pltpu.VMEM((bpw,), jnp.int32),
            pltpu.VMEM((bpw,), jnp.int32),
            pltpu.VMEM((bpw,), jnp.int32),
            pltpu.VMEM((bpw, DW), jnp.float32),
            pltpu.VMEM((bpw, DW), jnp.float32),
            pltpu.VMEM((bpw,), jnp.float32),
            pltpu.VMEM((bpw,), jnp.float32),
            pltpu.SemaphoreType.DMA,
            pltpu.SemaphoreType.DMA,
            pltpu.SemaphoreType.DMA,
            pltpu.SemaphoreType.DMA,
        ],
    )
    def gather_kernel(
        user_emb_hbm,
        item_emb_hbm,
        user_bias_hbm,
        item_bias_hbm,
        uids_hbm,
        iids_hbm,
        u_out,
        q_out,
        ub_out,
        ib_out,
        uvrow_v,
        ivrow_v,
        uidx_v,
        iidx_v,
        urows_v,
        qrows_v,
        ubr_v,
        ibr_v,
        sem_u,
        sem_q,
        sem_ub,
        sem_ib,
    ):
        wid = lax.axis_index("s") * nc + lax.axis_index("c")
        base = wid * bpw
        pltpu.sync_copy(uids_hbm.at[pl.ds(base, bpw)], uidx_v)
        pltpu.sync_copy(iids_hbm.at[pl.ds(base, bpw)], iidx_v)
        for k in range(bpw // 16):
            sl = pl.ds(k * 16, 16)
            uvrow_v[sl] = lax.shift_right_logical(uidx_v[sl], 2)
            ivrow_v[sl] = lax.shift_right_logical(iidx_v[sl], 2)
        cu = pltpu.async_copy(user_emb_hbm.at[uvrow_v], urows_v, sem_u)
        cq = pltpu.async_copy(item_emb_hbm.at[ivrow_v], qrows_v, sem_q)
        cub = pltpu.async_copy(user_bias_hbm.at[uidx_v], ubr_v, sem_ub)
        cib = pltpu.async_copy(item_bias_hbm.at[iidx_v], ibr_v, sem_ib)
        cu.wait()
        cq.wait()
        cub.wait()
        cib.wait()
        pltpu.sync_copy(urows_v, u_out.at[pl.ds(base, bpw)])
        pltpu.sync_copy(qrows_v, q_out.at[pl.ds(base, bpw)])
        pltpu.sync_copy(ubr_v, ub_out.at[pl.ds(base, bpw)])
        pltpu.sync_copy(ibr_v, ib_out.at[pl.ds(base, bpw)])

    return gather_kernel


def _select_chunk(rows, ids):
    sel = lax.rem(ids, jnp.int32(PK))
    out = jnp.zeros((B, D), dtype=jnp.float32)
    for k in range(PK):
        out = jnp.where(sel == k, rows[:, k * D:(k + 1) * D], out)
    return out


def _tc_body(u4_ref, q4_ref, ub_ref, ib_ref, uid_ref, iid_ref,
             w1t_ref, b1_ref, w2t_ref, b2_ref, preds_ref, score_ref):
    u = _select_chunk(u4_ref[...], uid_ref[...])
    q = _select_chunk(q4_ref[...], iid_ref[...])
    uq = u * q
    bias_col = ub_ref[...] + ib_ref[...]
    ones = jnp.ones((B, D), dtype=jnp.float32)
    preds = lax.dot_general(
        ones, uq, (((1,), (1,)), ((), ())),
        preferred_element_type=jnp.float32,
    )
    preds_ref[...] = preds + bias_col
    cat = jnp.concatenate([u, q, uq], axis=1)
    h = lax.dot_general(
        cat, w1t_ref[...], (((1,), (0,)), ((), ())),
        preferred_element_type=jnp.float32,
    )
    h = jnp.maximum(h + b1_ref[...], 0.0)
    s = lax.dot_general(
        h, w2t_ref[...], (((1,), (0,)), ((), ())),
        preferred_element_type=jnp.float32,
    )
    score_ref[...] = jnp.maximum(s + b2_ref[...], 0.0)


_sc_gather = None


def kernel(user_emb, item_emb, user_bias, item_bias, W1, bias1, W2, bias2,
           user_ids, item_ids):
    global _sc_gather
    if _sc_gather is None:
        _sc_gather = _make_sc_gather()

    uids = user_ids.astype(jnp.int32)
    iids = item_ids.astype(jnp.int32)
    u4, q4, ub, ib = _sc_gather(
        user_emb.reshape(-1, DW), item_emb.reshape(-1, DW),
        user_bias.reshape(-1), item_bias.reshape(-1),
        uids, iids,
    )

    w1t = W1.T
    b1 = bias1.reshape(1, 64)
    w2t = W2.T
    b2 = bias2.reshape(1, 1)

    preds, score = pl.pallas_call(
        _tc_body,
        out_shape=(
            jax.ShapeDtypeStruct((B, B), jnp.float32),
            jax.ShapeDtypeStruct((B, 1), jnp.float32),
        ),
    )(u4, q4, ub.reshape(B, 1), ib.reshape(B, 1),
      uids.reshape(B, 1), iids.reshape(B, 1), w1t, b1, w2t, b2)
    return (preds, score)
